# initial kernel scaffold (unmeasured)
import jax
import jax.numpy as jnp
from jax import lax
from jax.experimental import pallas as pl
from jax.experimental.pallas import tpu as pltpu

N_DEV = 4
Q_CHUNK = 1024


def kernel(q, k, v):
    s_per, d = q.shape
    n_chunks = s_per // Q_CHUNK
    scale = 1.0 / (d ** 0.5)

    qb = q.astype(jnp.bfloat16)
    kb = k.astype(jnp.bfloat16)
    vb = v.astype(jnp.bfloat16)

    def body(q_ref, k_ref, v_ref, out_ref, kcomm, vcomm,
             ksend, krecv, vsend, vrecv, credit_sem):
        my = lax.axis_index("i")
        left = (my - 1) % N_DEV
        right = (my + 1) % N_DEV

        barrier = pltpu.get_barrier_semaphore()
        for nbr in (left, right):
            pl.semaphore_signal(barrier, inc=1, device_id=(nbr,),
                                device_id_type=pl.DeviceIdType.MESH)
        pl.semaphore_wait(barrier, 2)

        m = [jnp.full((Q_CHUNK, 1), -jnp.inf, jnp.float32)
             for _ in range(n_chunks)]
        l = [jnp.zeros((Q_CHUNK, 1), jnp.float32) for _ in range(n_chunks)]
        acc = [jnp.zeros((Q_CHUNK, d), jnp.float32) for _ in range(n_chunks)]

        for h in range(N_DEV):
            if h == 0:
                k_src, v_src = k_ref, v_ref
            else:
                k_src, v_src = kcomm.at[(h - 1) % 2], vcomm.at[(h - 1) % 2]

            if h < N_DEV - 1:
                if h >= 1:
                    pl.semaphore_wait(credit_sem, 1)
                k_rdma = pltpu.make_async_remote_copy(
                    src_ref=k_src, dst_ref=kcomm.at[h % 2],
                    send_sem=ksend.at[h], recv_sem=krecv.at[h],
                    device_id=(right,), device_id_type=pl.DeviceIdType.MESH,
                )
                v_rdma = pltpu.make_async_remote_copy(
                    src_ref=v_src, dst_ref=vcomm.at[h % 2],
                    send_sem=vsend.at[h], recv_sem=vrecv.at[h],
                    device_id=(right,), device_id_type=pl.DeviceIdType.MESH,
                )
                k_rdma.start()
                v_rdma.start()
                k_rdma.wait()
                v_rdma.wait()

            kh = k_src[...]
            vh = v_src[...]
            for c in range(n_chunks):
                qc = q_ref[pl.ds(c * Q_CHUNK, Q_CHUNK), :]
                s = lax.dot_general(
                    qc, kh, (((1,), (1,)), ((), ())),
                    preferred_element_type=jnp.float32,
                ) * scale
                m_new = jnp.maximum(m[c], jnp.max(s, axis=1, keepdims=True))
                p = jnp.exp(s - m_new)
                alpha = jnp.exp(m[c] - m_new)
                l[c] = l[c] * alpha + jnp.sum(p, axis=1, keepdims=True)
                acc[c] = acc[c] * alpha + lax.dot(
                    p.astype(jnp.bfloat16), vh,
                    preferred_element_type=jnp.float32,
                )
                m[c] = m_new

            if h < N_DEV - 1:
                pl.semaphore_signal(credit_sem, inc=1, device_id=(left,),
                                    device_id_type=pl.DeviceIdType.MESH)

        for c in range(n_chunks):
            out_ref[pl.ds(c * Q_CHUNK, Q_CHUNK), :] = acc[c] / l[c]

    return pl.pallas_call(
        body,
        out_shape=jax.ShapeDtypeStruct((s_per, d), jnp.float32),
        in_specs=[pl.BlockSpec(memory_space=pltpu.VMEM)] * 3,
        out_specs=pl.BlockSpec(memory_space=pltpu.VMEM),
        scratch_shapes=[
            pltpu.VMEM((2, s_per, d), jnp.bfloat16),
            pltpu.VMEM((2, s_per, d), jnp.bfloat16),
            pltpu.SemaphoreType.DMA((N_DEV - 1,)),
            pltpu.SemaphoreType.DMA((N_DEV - 1,)),
            pltpu.SemaphoreType.DMA((N_DEV - 1,)),
            pltpu.SemaphoreType.DMA((N_DEV - 1,)),
            pltpu.SemaphoreType.REGULAR,
        ],
        compiler_params=pltpu.CompilerParams(
            collective_id=0,
            vmem_limit_bytes=128 * 1024 * 1024,
        ),
    )(qb, kb, vb)


# baseline (device time: 567534 ns/iter reference)
import jax
import jax.numpy as jnp
from jax import lax
from jax.experimental import pallas as pl
from jax.experimental.pallas import tpu as pltpu

N_DEV = 4
Q_TILE = 1024
K_TILE = 1024


def kernel(q, k, v):
    s_per, d = q.shape
    n_q = s_per // Q_TILE
    n_k = s_per // K_TILE
    scale = 1.0 / (d ** 0.5)

    qb = (q * scale).astype(jnp.bfloat16)
    kb = k.astype(jnp.bfloat16)
    vb = v.astype(jnp.bfloat16)

    def body(q_ref, k_ref, v_ref, out_ref, kcomm, vcomm, m_ref, l_ref,
             ksend, krecv, vsend, vrecv, credit_sem):
        my = lax.axis_index("i")
        left = (my - 1) % N_DEV
        right = (my + 1) % N_DEV

        barrier = pltpu.get_barrier_semaphore()
        for nbr in (left, right):
            pl.semaphore_signal(barrier, inc=1, device_id=(nbr,),
                                device_id_type=pl.DeviceIdType.MESH)
        pl.semaphore_wait(barrier, 2)

        m_ref[...] = jnp.full((s_per, 1), -jnp.inf, jnp.float32)
        l_ref[...] = jnp.zeros((s_per, 1), jnp.float32)
        out_ref[...] = jnp.zeros((s_per, d), jnp.float32)

        for h in range(N_DEV):
            if h == 0:
                k_src, v_src = k_ref, v_ref
            else:
                k_src, v_src = kcomm.at[(h - 1) % 2], vcomm.at[(h - 1) % 2]

            if h < N_DEV - 1:
                if h >= 1:
                    pl.semaphore_wait(credit_sem, 1)
                k_rdma = pltpu.make_async_remote_copy(
                    src_ref=k_src, dst_ref=kcomm.at[h % 2],
                    send_sem=ksend.at[h], recv_sem=krecv.at[h],
                    device_id=(right,), device_id_type=pl.DeviceIdType.MESH,
                )
                v_rdma = pltpu.make_async_remote_copy(
                    src_ref=v_src, dst_ref=vcomm.at[h % 2],
                    send_sem=vsend.at[h], recv_sem=vrecv.at[h],
                    device_id=(right,), device_id_type=pl.DeviceIdType.MESH,
                )
                k_rdma.start()
                v_rdma.start()
                k_rdma.wait()
                v_rdma.wait()

            def q_loop(c, _, k_src=k_src, v_src=v_src):
                qs = pl.ds(c * Q_TILE, Q_TILE)
                qc = q_ref[qs, :]

                def k_loop(kc, _):
                    ks = pl.ds(kc * K_TILE, K_TILE)
                    s = lax.dot_general(
                        qc, k_src[ks, :], (((1,), (1,)), ((), ())),
                        preferred_element_type=jnp.float32,
                    )
                    m_old = m_ref[qs, :]
                    m_new = jnp.maximum(m_old, jnp.max(s, axis=1,
                                                       keepdims=True))
                    p = jnp.exp(s - m_new)
                    alpha = jnp.exp(m_old - m_new)
                    l_ref[qs, :] = l_ref[qs, :] * alpha + jnp.sum(
                        p, axis=1, keepdims=True)
                    out_ref[qs, :] = out_ref[qs, :] * alpha + lax.dot(
                        p.astype(jnp.bfloat16), v_src[ks, :],
                        preferred_element_type=jnp.float32,
                    )
                    m_ref[qs, :] = m_new
                    return 0

                return lax.fori_loop(0, n_k, k_loop, 0)

            lax.fori_loop(0, n_q, q_loop, 0)

            if h < N_DEV - 2:
                pl.semaphore_signal(credit_sem, inc=1, device_id=(left,),
                                    device_id_type=pl.DeviceIdType.MESH)

        out_ref[...] = out_ref[...] / l_ref[...]

    return pl.pallas_call(
        body,
        out_shape=jax.ShapeDtypeStruct((s_per, d), jnp.float32),
        in_specs=[pl.BlockSpec(memory_space=pltpu.VMEM)] * 3,
        out_specs=pl.BlockSpec(memory_space=pltpu.VMEM),
        scratch_shapes=[
            pltpu.VMEM((2, s_per, d), jnp.bfloat16),
            pltpu.VMEM((2, s_per, d), jnp.bfloat16),
            pltpu.VMEM((s_per, 1), jnp.float32),
            pltpu.VMEM((s_per, 1), jnp.float32),
            pltpu.SemaphoreType.DMA((N_DEV - 1,)),
            pltpu.SemaphoreType.DMA((N_DEV - 1,)),
            pltpu.SemaphoreType.DMA((N_DEV - 1,)),
            pltpu.SemaphoreType.DMA((N_DEV - 1,)),
            pltpu.SemaphoreType.REGULAR,
        ],
        compiler_params=pltpu.CompilerParams(
            collective_id=0,
            vmem_limit_bytes=64 * 1024 * 1024,
        ),
    )(qb, kb, vb)


# device time: 373526 ns/iter; 1.5194x vs baseline; 1.5194x over previous
import jax
import jax.numpy as jnp
from jax import lax
from jax.experimental import pallas as pl
from jax.experimental.pallas import tpu as pltpu

N_DEV = 4
Q_TILE = 1024
K_TILE = 1024


def kernel(q, k, v):
    s_per, d = q.shape
    n_q = s_per // Q_TILE
    n_k = s_per // K_TILE
    scale = 1.0 / (d ** 0.5)

    qb = (q * scale).astype(jnp.bfloat16)
    kb = k.astype(jnp.bfloat16)
    vb = v.astype(jnp.bfloat16)

    def body(q_ref, k_ref, v_ref, out_ref, kcomm, vcomm, m_ref, l_ref,
             ksend, krecv, vsend, vrecv, credit_sem):
        my = lax.axis_index("i")
        left = (my - 1) % N_DEV
        right = (my + 1) % N_DEV

        barrier = pltpu.get_barrier_semaphore()
        for nbr in (left, right):
            pl.semaphore_signal(barrier, inc=1, device_id=(nbr,),
                                device_id_type=pl.DeviceIdType.MESH)
        pl.semaphore_wait(barrier, 2)

        m_ref[...] = jnp.full((s_per, 1), -jnp.inf, jnp.float32)
        l_ref[...] = jnp.zeros((s_per, 1), jnp.float32)
        out_ref[...] = jnp.zeros((s_per, d), jnp.float32)

        for h in range(N_DEV):
            if h == 0:
                k_src, v_src = k_ref, v_ref
            else:
                k_src, v_src = kcomm.at[(h - 1) % 2], vcomm.at[(h - 1) % 2]

            if h < N_DEV - 1:
                if h >= 1:
                    pl.semaphore_wait(credit_sem, 1)
                k_rdma = pltpu.make_async_remote_copy(
                    src_ref=k_src, dst_ref=kcomm.at[h % 2],
                    send_sem=ksend.at[h], recv_sem=krecv.at[h],
                    device_id=(right,), device_id_type=pl.DeviceIdType.MESH,
                )
                v_rdma = pltpu.make_async_remote_copy(
                    src_ref=v_src, dst_ref=vcomm.at[h % 2],
                    send_sem=vsend.at[h], recv_sem=vrecv.at[h],
                    device_id=(right,), device_id_type=pl.DeviceIdType.MESH,
                )
                k_rdma.start()
                v_rdma.start()

            def q_loop(c, _, k_src=k_src, v_src=v_src):
                qs = pl.ds(c * Q_TILE, Q_TILE)
                qc = q_ref[qs, :]

                def k_loop(kc, _):
                    ks = pl.ds(kc * K_TILE, K_TILE)
                    s = lax.dot_general(
                        qc, k_src[ks, :], (((1,), (1,)), ((), ())),
                        preferred_element_type=jnp.float32,
                    )
                    m_old = m_ref[qs, :]
                    m_new = jnp.maximum(m_old, jnp.max(s, axis=1,
                                                       keepdims=True))
                    p = jnp.exp(s - m_new)
                    alpha = jnp.exp(m_old - m_new)
                    l_ref[qs, :] = l_ref[qs, :] * alpha + jnp.sum(
                        p, axis=1, keepdims=True)
                    out_ref[qs, :] = out_ref[qs, :] * alpha + lax.dot(
                        p.astype(jnp.bfloat16), v_src[ks, :],
                        preferred_element_type=jnp.float32,
                    )
                    m_ref[qs, :] = m_new
                    return 0

                return lax.fori_loop(0, n_k, k_loop, 0)

            lax.fori_loop(0, n_q, q_loop, 0)

            if h < N_DEV - 1:
                k_rdma.wait()
                v_rdma.wait()
            if h < N_DEV - 2:
                pl.semaphore_signal(credit_sem, inc=1, device_id=(left,),
                                    device_id_type=pl.DeviceIdType.MESH)

        out_ref[...] = out_ref[...] / l_ref[...]

    return pl.pallas_call(
        body,
        out_shape=jax.ShapeDtypeStruct((s_per, d), jnp.float32),
        in_specs=[pl.BlockSpec(memory_space=pltpu.VMEM)] * 3,
        out_specs=pl.BlockSpec(memory_space=pltpu.VMEM),
        scratch_shapes=[
            pltpu.VMEM((2, s_per, d), jnp.bfloat16),
            pltpu.VMEM((2, s_per, d), jnp.bfloat16),
            pltpu.VMEM((s_per, 1), jnp.float32),
            pltpu.VMEM((s_per, 1), jnp.float32),
            pltpu.SemaphoreType.DMA((N_DEV - 1,)),
            pltpu.SemaphoreType.DMA((N_DEV - 1,)),
            pltpu.SemaphoreType.DMA((N_DEV - 1,)),
            pltpu.SemaphoreType.DMA((N_DEV - 1,)),
            pltpu.SemaphoreType.REGULAR,
        ],
        compiler_params=pltpu.CompilerParams(
            collective_id=0,
            vmem_limit_bytes=64 * 1024 * 1024,
        ),
    )(qb, kb, vb)


# device time: 293558 ns/iter; 1.9333x vs baseline; 1.2724x over previous
import jax
import jax.numpy as jnp
from jax import lax
from jax.experimental import pallas as pl
from jax.experimental.pallas import tpu as pltpu

N_DEV = 4
Q_TILE = 1024
K_TILE = 1024


def kernel(q, k, v):
    s_per, d = q.shape
    half = s_per // 2
    n_q = s_per // Q_TILE
    scale = 1.0 / (d ** 0.5)

    qb = (q * scale).astype(jnp.bfloat16)
    kb = k.astype(jnp.bfloat16)
    vb = v.astype(jnp.bfloat16)

    def body(q_ref, k_ref, v_ref, out_ref,
             kcw, vcw, kccw, vccw, m_ref, l_ref,
             kcw_s, kcw_r, vcw_s, vcw_r,
             kccw_s, kccw_r, vccw_s, vccw_r,
             credit_cw, credit_ccw):
        my = lax.axis_index("i")
        left = (my - 1) % N_DEV
        right = (my + 1) % N_DEV

        barrier = pltpu.get_barrier_semaphore()
        for nbr in (left, right):
            pl.semaphore_signal(barrier, inc=1, device_id=(nbr,),
                                device_id_type=pl.DeviceIdType.MESH)
        pl.semaphore_wait(barrier, 2)

        m_ref[...] = jnp.full((s_per, 1), -jnp.inf, jnp.float32)
        l_ref[...] = jnp.zeros((s_per, 1), jnp.float32)
        out_ref[...] = jnp.zeros((s_per, d), jnp.float32)

        def flash_update(k_src, v_src, n_kt):
            def q_loop(c, _):
                qs = pl.ds(c * Q_TILE, Q_TILE)
                qc = q_ref[qs, :]

                def k_loop(kc, _):
                    ks = pl.ds(kc * K_TILE, K_TILE)
                    s = lax.dot_general(
                        qc, k_src[ks, :], (((1,), (1,)), ((), ())),
                        preferred_element_type=jnp.float32,
                    )
                    m_old = m_ref[qs, :]
                    m_new = jnp.maximum(m_old, jnp.max(s, axis=1,
                                                       keepdims=True))
                    p = jnp.exp(s - m_new)
                    alpha = jnp.exp(m_old - m_new)
                    l_ref[qs, :] = l_ref[qs, :] * alpha + jnp.sum(
                        p, axis=1, keepdims=True)
                    out_ref[qs, :] = out_ref[qs, :] * alpha + lax.dot(
                        p.astype(jnp.bfloat16), v_src[ks, :],
                        preferred_element_type=jnp.float32,
                    )
                    m_ref[qs, :] = m_new
                    return 0

                return lax.fori_loop(0, n_kt, k_loop, 0)

            lax.fori_loop(0, n_q, q_loop, 0)

        for h in range(N_DEV):
            if h == 0:
                cw_k, cw_v = k_ref.at[pl.ds(0, half)], v_ref.at[pl.ds(0, half)]
                ccw_k, ccw_v = k_ref.at[pl.ds(half, half)], v_ref.at[pl.ds(half, half)]
            else:
                prev = (h - 1) % 2
                cw_k, cw_v = kcw.at[prev], vcw.at[prev]
                ccw_k, ccw_v = kccw.at[prev], vccw.at[prev]

            rdmas = []
            if h < N_DEV - 1:
                if h >= 1:
                    pl.semaphore_wait(credit_cw, 1)
                    pl.semaphore_wait(credit_ccw, 1)
                slot = h % 2
                for src, dst, sems_s, sems_r, dev in (
                    (cw_k, kcw.at[slot], kcw_s, kcw_r, right),
                    (cw_v, vcw.at[slot], vcw_s, vcw_r, right),
                    (ccw_k, kccw.at[slot], kccw_s, kccw_r, left),
                    (ccw_v, vccw.at[slot], vccw_s, vccw_r, left),
                ):
                    rdma = pltpu.make_async_remote_copy(
                        src_ref=src, dst_ref=dst,
                        send_sem=sems_s.at[h], recv_sem=sems_r.at[h],
                        device_id=(dev,),
                        device_id_type=pl.DeviceIdType.MESH,
                    )
                    rdma.start()
                    rdmas.append(rdma)

            if h == 0:
                flash_update(k_ref, v_ref, s_per // K_TILE)
            else:
                flash_update(cw_k, cw_v, half // K_TILE)
                flash_update(ccw_k, ccw_v, half // K_TILE)

            for rdma in rdmas:
                rdma.wait()
            if h < N_DEV - 2:
                pl.semaphore_signal(credit_cw, inc=1, device_id=(left,),
                                    device_id_type=pl.DeviceIdType.MESH)
                pl.semaphore_signal(credit_ccw, inc=1, device_id=(right,),
                                    device_id_type=pl.DeviceIdType.MESH)

        out_ref[...] = out_ref[...] / l_ref[...]

    return pl.pallas_call(
        body,
        out_shape=jax.ShapeDtypeStruct((s_per, d), jnp.float32),
        in_specs=[pl.BlockSpec(memory_space=pltpu.VMEM)] * 3,
        out_specs=pl.BlockSpec(memory_space=pltpu.VMEM),
        scratch_shapes=[
            pltpu.VMEM((2, half, d), jnp.bfloat16),
            pltpu.VMEM((2, half, d), jnp.bfloat16),
            pltpu.VMEM((2, half, d), jnp.bfloat16),
            pltpu.VMEM((2, half, d), jnp.bfloat16),
            pltpu.VMEM((s_per, 1), jnp.float32),
            pltpu.VMEM((s_per, 1), jnp.float32),
            pltpu.SemaphoreType.DMA((N_DEV - 1,)),
            pltpu.SemaphoreType.DMA((N_DEV - 1,)),
            pltpu.SemaphoreType.DMA((N_DEV - 1,)),
            pltpu.SemaphoreType.DMA((N_DEV - 1,)),
            pltpu.SemaphoreType.DMA((N_DEV - 1,)),
            pltpu.SemaphoreType.DMA((N_DEV - 1,)),
            pltpu.SemaphoreType.DMA((N_DEV - 1,)),
            pltpu.SemaphoreType.DMA((N_DEV - 1,)),
            pltpu.SemaphoreType.REGULAR,
            pltpu.SemaphoreType.REGULAR,
        ],
        compiler_params=pltpu.CompilerParams(
            collective_id=0,
            vmem_limit_bytes=64 * 1024 * 1024,
        ),
    )(qb, kb, vb)


# device time: 274503 ns/iter; 2.0675x vs baseline; 1.0694x over previous
import jax
import jax.numpy as jnp
from jax import lax
from jax.experimental import pallas as pl
from jax.experimental.pallas import tpu as pltpu

N_DEV = 4
Q_TILE = 1024
K_TILE = 1024


def kernel(q, k, v):
    s_per, d = q.shape
    half = s_per // 2
    n_q = s_per // Q_TILE
    scale = 1.0 / (d ** 0.5)

    qb = (q * scale).astype(jnp.bfloat16)
    kb = k.astype(jnp.bfloat16)
    vb = v.astype(jnp.bfloat16)

    def body(q_ref, k_ref, v_ref, out_ref,
             kcw, vcw, kccw, vccw, m_ref, l_ref,
             kcw_s, kcw_r, vcw_s, vcw_r,
             kccw_s, kccw_r, vccw_s, vccw_r,
             credit_cw, credit_ccw):
        my = lax.axis_index("i")
        left = (my - 1) % N_DEV
        right = (my + 1) % N_DEV

        barrier = pltpu.get_barrier_semaphore()
        for nbr in (left, right):
            pl.semaphore_signal(barrier, inc=1, device_id=(nbr,),
                                device_id_type=pl.DeviceIdType.MESH)
        pl.semaphore_wait(barrier, 2)

        m_ref[...] = jnp.full((s_per, 1), -jnp.inf, jnp.float32)
        l_ref[...] = jnp.zeros((s_per, 1), jnp.float32)
        out_ref[...] = jnp.zeros((s_per, d), jnp.float32)

        def flash_update(k_src, v_src, n_kt):
            def q_loop(c, _):
                qs = pl.ds(c * Q_TILE, Q_TILE)
                qc = q_ref[qs, :]

                for kc in range(n_kt):
                    ks = pl.ds(kc * K_TILE, K_TILE)
                    s = lax.dot_general(
                        qc, k_src[ks, :], (((1,), (1,)), ((), ())),
                        preferred_element_type=jnp.float32,
                    )
                    m_old = m_ref[qs, :]
                    m_new = jnp.maximum(m_old, jnp.max(s, axis=1,
                                                       keepdims=True))
                    p = jnp.exp(s - m_new)
                    alpha = jnp.exp(m_old - m_new)
                    l_ref[qs, :] = l_ref[qs, :] * alpha + jnp.sum(
                        p, axis=1, keepdims=True)
                    out_ref[qs, :] = out_ref[qs, :] * alpha + lax.dot(
                        p.astype(jnp.bfloat16), v_src[ks, :],
                        preferred_element_type=jnp.float32,
                    )
                    m_ref[qs, :] = m_new
                return 0

            lax.fori_loop(0, n_q, q_loop, 0)

        for h in range(N_DEV):
            if h == 0:
                cw_k, cw_v = k_ref.at[pl.ds(0, half)], v_ref.at[pl.ds(0, half)]
                ccw_k, ccw_v = k_ref.at[pl.ds(half, half)], v_ref.at[pl.ds(half, half)]
            else:
                prev = (h - 1) % 2
                cw_k, cw_v = kcw.at[prev], vcw.at[prev]
                ccw_k, ccw_v = kccw.at[prev], vccw.at[prev]

            rdmas = []
            if h < N_DEV - 1:
                if h >= 1:
                    pl.semaphore_wait(credit_cw, 1)
                    pl.semaphore_wait(credit_ccw, 1)
                slot = h % 2
                for src, dst, sems_s, sems_r, dev in (
                    (cw_k, kcw.at[slot], kcw_s, kcw_r, right),
                    (cw_v, vcw.at[slot], vcw_s, vcw_r, right),
                    (ccw_k, kccw.at[slot], kccw_s, kccw_r, left),
                    (ccw_v, vccw.at[slot], vccw_s, vccw_r, left),
                ):
                    rdma = pltpu.make_async_remote_copy(
                        src_ref=src, dst_ref=dst,
                        send_sem=sems_s.at[h], recv_sem=sems_r.at[h],
                        device_id=(dev,),
                        device_id_type=pl.DeviceIdType.MESH,
                    )
                    rdma.start()
                    rdmas.append(rdma)

            if h == 0:
                flash_update(k_ref, v_ref, s_per // K_TILE)
            else:
                flash_update(cw_k, cw_v, half // K_TILE)
                flash_update(ccw_k, ccw_v, half // K_TILE)

            for rdma in rdmas:
                rdma.wait()
            if h < N_DEV - 2:
                pl.semaphore_signal(credit_cw, inc=1, device_id=(left,),
                                    device_id_type=pl.DeviceIdType.MESH)
                pl.semaphore_signal(credit_ccw, inc=1, device_id=(right,),
                                    device_id_type=pl.DeviceIdType.MESH)

        out_ref[...] = out_ref[...] / l_ref[...]

    return pl.pallas_call(
        body,
        out_shape=jax.ShapeDtypeStruct((s_per, d), jnp.float32),
        in_specs=[pl.BlockSpec(memory_space=pltpu.VMEM)] * 3,
        out_specs=pl.BlockSpec(memory_space=pltpu.VMEM),
        scratch_shapes=[
            pltpu.VMEM((2, half, d), jnp.bfloat16),
            pltpu.VMEM((2, half, d), jnp.bfloat16),
            pltpu.VMEM((2, half, d), jnp.bfloat16),
            pltpu.VMEM((2, half, d), jnp.bfloat16),
            pltpu.VMEM((s_per, 1), jnp.float32),
            pltpu.VMEM((s_per, 1), jnp.float32),
            pltpu.SemaphoreType.DMA((N_DEV - 1,)),
            pltpu.SemaphoreType.DMA((N_DEV - 1,)),
            pltpu.SemaphoreType.DMA((N_DEV - 1,)),
            pltpu.SemaphoreType.DMA((N_DEV - 1,)),
            pltpu.SemaphoreType.DMA((N_DEV - 1,)),
            pltpu.SemaphoreType.DMA((N_DEV - 1,)),
            pltpu.SemaphoreType.DMA((N_DEV - 1,)),
            pltpu.SemaphoreType.DMA((N_DEV - 1,)),
            pltpu.SemaphoreType.REGULAR,
            pltpu.SemaphoreType.REGULAR,
        ],
        compiler_params=pltpu.CompilerParams(
            collective_id=0,
            vmem_limit_bytes=64 * 1024 * 1024,
        ),
    )(qb, kb, vb)


# device time: 270151 ns/iter; 2.1008x vs baseline; 1.0161x over previous
import jax
import jax.numpy as jnp
from jax import lax
from jax.experimental import pallas as pl
from jax.experimental.pallas import tpu as pltpu

N_DEV = 4
Q_TILE = 1024
K_TILE = 1024


def kernel(q, k, v):
    s_per, d = q.shape
    half = s_per // 2
    n_q = s_per // Q_TILE
    scale = 1.0 / (d ** 0.5)

    qb = (q * scale).astype(jnp.bfloat16)
    kb = k.astype(jnp.bfloat16)
    vb = v.astype(jnp.bfloat16)

    def body(q_ref, k_ref, v_ref, out_ref,
             kcw, vcw, kccw, vccw, m_ref, l_ref,
             kcw_s, kcw_r, vcw_s, vcw_r,
             kccw_s, kccw_r, vccw_s, vccw_r,
             credit_cw, credit_ccw):
        my = lax.axis_index("i")
        left = (my - 1) % N_DEV
        right = (my + 1) % N_DEV

        barrier = pltpu.get_barrier_semaphore()
        for nbr in (left, right):
            pl.semaphore_signal(barrier, inc=1, device_id=(nbr,),
                                device_id_type=pl.DeviceIdType.MESH)
        pl.semaphore_wait(barrier, 2)

        m_ref[...] = jnp.full((s_per, 1), -jnp.inf, jnp.float32)
        l_ref[...] = jnp.zeros((s_per, 1), jnp.float32)
        out_ref[...] = jnp.zeros((s_per, d), jnp.float32)

        def flash_update(k_src, v_src, n_kt):
            def q_loop(c, _):
                qs = pl.ds(c * Q_TILE, Q_TILE)
                qc = q_ref[qs, :]

                m_old = m_ref[qs, :]
                l_acc = l_ref[qs, :]
                acc = out_ref[qs, :]
                for kc in range(n_kt):
                    ks = pl.ds(kc * K_TILE, K_TILE)
                    s = lax.dot_general(
                        qc, k_src[ks, :], (((1,), (1,)), ((), ())),
                        preferred_element_type=jnp.float32,
                    )
                    m_new = jnp.maximum(m_old, jnp.max(s, axis=1,
                                                       keepdims=True))
                    p = jnp.exp(s - m_new)
                    alpha = jnp.exp(m_old - m_new)
                    l_acc = l_acc * alpha + jnp.sum(p, axis=1,
                                                    keepdims=True)
                    acc = acc * alpha + lax.dot(
                        p.astype(jnp.bfloat16), v_src[ks, :],
                        preferred_element_type=jnp.float32,
                    )
                    m_old = m_new
                m_ref[qs, :] = m_old
                l_ref[qs, :] = l_acc
                out_ref[qs, :] = acc
                return 0

            lax.fori_loop(0, n_q, q_loop, 0)

        for h in range(N_DEV):
            if h == 0:
                cw_k, cw_v = k_ref.at[pl.ds(0, half)], v_ref.at[pl.ds(0, half)]
                ccw_k, ccw_v = k_ref.at[pl.ds(half, half)], v_ref.at[pl.ds(half, half)]
            else:
                prev = (h - 1) % 2
                cw_k, cw_v = kcw.at[prev], vcw.at[prev]
                ccw_k, ccw_v = kccw.at[prev], vccw.at[prev]

            rdmas = []
            if h < N_DEV - 1:
                if h >= 1:
                    pl.semaphore_wait(credit_cw, 1)
                    pl.semaphore_wait(credit_ccw, 1)
                slot = h % 2
                for src, dst, sems_s, sems_r, dev in (
                    (cw_k, kcw.at[slot], kcw_s, kcw_r, right),
                    (cw_v, vcw.at[slot], vcw_s, vcw_r, right),
                    (ccw_k, kccw.at[slot], kccw_s, kccw_r, left),
                    (ccw_v, vccw.at[slot], vccw_s, vccw_r, left),
                ):
                    rdma = pltpu.make_async_remote_copy(
                        src_ref=src, dst_ref=dst,
                        send_sem=sems_s.at[h], recv_sem=sems_r.at[h],
                        device_id=(dev,),
                        device_id_type=pl.DeviceIdType.MESH,
                    )
                    rdma.start()
                    rdmas.append(rdma)

            if h == 0:
                flash_update(k_ref, v_ref, s_per // K_TILE)
            else:
                flash_update(cw_k, cw_v, half // K_TILE)
                flash_update(ccw_k, ccw_v, half // K_TILE)

            for rdma in rdmas:
                rdma.wait()
            if h < N_DEV - 2:
                pl.semaphore_signal(credit_cw, inc=1, device_id=(left,),
                                    device_id_type=pl.DeviceIdType.MESH)
                pl.semaphore_signal(credit_ccw, inc=1, device_id=(right,),
                                    device_id_type=pl.DeviceIdType.MESH)

        out_ref[...] = out_ref[...] / l_ref[...]

    return pl.pallas_call(
        body,
        out_shape=jax.ShapeDtypeStruct((s_per, d), jnp.float32),
        in_specs=[pl.BlockSpec(memory_space=pltpu.VMEM)] * 3,
        out_specs=pl.BlockSpec(memory_space=pltpu.VMEM),
        scratch_shapes=[
            pltpu.VMEM((2, half, d), jnp.bfloat16),
            pltpu.VMEM((2, half, d), jnp.bfloat16),
            pltpu.VMEM((2, half, d), jnp.bfloat16),
            pltpu.VMEM((2, half, d), jnp.bfloat16),
            pltpu.VMEM((s_per, 1), jnp.float32),
            pltpu.VMEM((s_per, 1), jnp.float32),
            pltpu.SemaphoreType.DMA((N_DEV - 1,)),
            pltpu.SemaphoreType.DMA((N_DEV - 1,)),
            pltpu.SemaphoreType.DMA((N_DEV - 1,)),
            pltpu.SemaphoreType.DMA((N_DEV - 1,)),
            pltpu.SemaphoreType.DMA((N_DEV - 1,)),
            pltpu.SemaphoreType.DMA((N_DEV - 1,)),
            pltpu.SemaphoreType.DMA((N_DEV - 1,)),
            pltpu.SemaphoreType.DMA((N_DEV - 1,)),
            pltpu.SemaphoreType.REGULAR,
            pltpu.SemaphoreType.REGULAR,
        ],
        compiler_params=pltpu.CompilerParams(
            collective_id=0,
            vmem_limit_bytes=64 * 1024 * 1024,
        ),
    )(qb, kb, vb)


# device time: 221229 ns/iter; 2.5654x vs baseline; 1.2211x over previous
import jax
import jax.numpy as jnp
from jax import lax
from jax.experimental import pallas as pl
from jax.experimental.pallas import tpu as pltpu

N_DEV = 4
Q_TILE = 1024
K_TILE = 1024


def kernel(q, k, v):
    s_per, d = q.shape
    half = s_per // 2
    n_q = s_per // Q_TILE
    scale = 1.0 / (d ** 0.5)

    qb = (q * scale).astype(jnp.bfloat16)
    kb = k.astype(jnp.bfloat16)
    vb = v.astype(jnp.bfloat16)

    def body(q_ref, k_ref, v_ref, out_ref,
             kcw, vcw, kccw, vccw, l_ref,
             kcw_s, kcw_r, vcw_s, vcw_r,
             kccw_s, kccw_r, vccw_s, vccw_r,
             credit_cw, credit_ccw):
        my = lax.axis_index("i")
        left = (my - 1) % N_DEV
        right = (my + 1) % N_DEV

        barrier = pltpu.get_barrier_semaphore()
        for nbr in (left, right):
            pl.semaphore_signal(barrier, inc=1, device_id=(nbr,),
                                device_id_type=pl.DeviceIdType.MESH)
        pl.semaphore_wait(barrier, 2)

        l_ref[...] = jnp.zeros((s_per, 128), jnp.float32)
        out_ref[...] = jnp.zeros((s_per, d), jnp.float32)

        ones_b = jnp.ones((K_TILE, 128), jnp.bfloat16)

        def attn_update(k_src, v_src, n_kt):
            def q_loop(c, _):
                qs = pl.ds(c * Q_TILE, Q_TILE)
                qc = q_ref[qs, :]

                l_acc = l_ref[qs, :]
                acc = out_ref[qs, :]
                for kc in range(n_kt):
                    ks = pl.ds(kc * K_TILE, K_TILE)
                    s = lax.dot_general(
                        qc, k_src[ks, :], (((1,), (1,)), ((), ())),
                        preferred_element_type=jnp.float32,
                    )
                    p = jnp.exp(s).astype(jnp.bfloat16)
                    l_acc = l_acc + lax.dot(
                        p, ones_b, preferred_element_type=jnp.float32)
                    acc = acc + lax.dot(
                        p, v_src[ks, :],
                        preferred_element_type=jnp.float32,
                    )
                l_ref[qs, :] = l_acc
                out_ref[qs, :] = acc
                return 0

            lax.fori_loop(0, n_q, q_loop, 0)

        for h in range(N_DEV):
            if h == 0:
                cw_k, cw_v = k_ref.at[pl.ds(0, half)], v_ref.at[pl.ds(0, half)]
                ccw_k, ccw_v = k_ref.at[pl.ds(half, half)], v_ref.at[pl.ds(half, half)]
            else:
                prev = (h - 1) % 2
                cw_k, cw_v = kcw.at[prev], vcw.at[prev]
                ccw_k, ccw_v = kccw.at[prev], vccw.at[prev]

            rdmas = []
            if h < N_DEV - 1:
                if h >= 1:
                    pl.semaphore_wait(credit_cw, 1)
                    pl.semaphore_wait(credit_ccw, 1)
                slot = h % 2
                for src, dst, sems_s, sems_r, dev in (
                    (cw_k, kcw.at[slot], kcw_s, kcw_r, right),
                    (cw_v, vcw.at[slot], vcw_s, vcw_r, right),
                    (ccw_k, kccw.at[slot], kccw_s, kccw_r, left),
                    (ccw_v, vccw.at[slot], vccw_s, vccw_r, left),
                ):
                    rdma = pltpu.make_async_remote_copy(
                        src_ref=src, dst_ref=dst,
                        send_sem=sems_s.at[h], recv_sem=sems_r.at[h],
                        device_id=(dev,),
                        device_id_type=pl.DeviceIdType.MESH,
                    )
                    rdma.start()
                    rdmas.append(rdma)

            if h == 0:
                attn_update(k_ref, v_ref, s_per // K_TILE)
            else:
                attn_update(cw_k, cw_v, half // K_TILE)
                attn_update(ccw_k, ccw_v, half // K_TILE)

            for rdma in rdmas:
                rdma.wait()
            if h < N_DEV - 2:
                pl.semaphore_signal(credit_cw, inc=1, device_id=(left,),
                                    device_id_type=pl.DeviceIdType.MESH)
                pl.semaphore_signal(credit_ccw, inc=1, device_id=(right,),
                                    device_id_type=pl.DeviceIdType.MESH)

        out_ref[...] = out_ref[...] / l_ref[:, 0:1]

    return pl.pallas_call(
        body,
        out_shape=jax.ShapeDtypeStruct((s_per, d), jnp.float32),
        in_specs=[pl.BlockSpec(memory_space=pltpu.VMEM)] * 3,
        out_specs=pl.BlockSpec(memory_space=pltpu.VMEM),
        scratch_shapes=[
            pltpu.VMEM((2, half, d), jnp.bfloat16),
            pltpu.VMEM((2, half, d), jnp.bfloat16),
            pltpu.VMEM((2, half, d), jnp.bfloat16),
            pltpu.VMEM((2, half, d), jnp.bfloat16),
            pltpu.VMEM((s_per, 128), jnp.float32),
            pltpu.SemaphoreType.DMA((N_DEV - 1,)),
            pltpu.SemaphoreType.DMA((N_DEV - 1,)),
            pltpu.SemaphoreType.DMA((N_DEV - 1,)),
            pltpu.SemaphoreType.DMA((N_DEV - 1,)),
            pltpu.SemaphoreType.DMA((N_DEV - 1,)),
            pltpu.SemaphoreType.DMA((N_DEV - 1,)),
            pltpu.SemaphoreType.DMA((N_DEV - 1,)),
            pltpu.SemaphoreType.DMA((N_DEV - 1,)),
            pltpu.SemaphoreType.REGULAR,
            pltpu.SemaphoreType.REGULAR,
        ],
        compiler_params=pltpu.CompilerParams(
            collective_id=0,
            vmem_limit_bytes=64 * 1024 * 1024,
        ),
    )(qb, kb, vb)


# device time: 219491 ns/iter; 2.5857x vs baseline; 1.0079x over previous
import jax
import jax.numpy as jnp
from jax import lax
from jax.experimental import pallas as pl
from jax.experimental.pallas import tpu as pltpu

N_DEV = 4
Q_TILE = 1024
K_TILE = 2048


def kernel(q, k, v):
    s_per, d = q.shape
    half = s_per // 2
    n_q = s_per // Q_TILE
    scale = 1.0 / (d ** 0.5)

    qb = (q * scale).astype(jnp.bfloat16)
    kb = k.astype(jnp.bfloat16)
    vb = v.astype(jnp.bfloat16)

    def body(q_ref, k_ref, v_ref, out_ref,
             kcw, vcw, kccw, vccw, l_ref,
             kcw_s, kcw_r, vcw_s, vcw_r,
             kccw_s, kccw_r, vccw_s, vccw_r,
             credit_cw, credit_ccw):
        my = lax.axis_index("i")
        left = (my - 1) % N_DEV
        right = (my + 1) % N_DEV

        barrier = pltpu.get_barrier_semaphore()
        for nbr in (left, right):
            pl.semaphore_signal(barrier, inc=1, device_id=(nbr,),
                                device_id_type=pl.DeviceIdType.MESH)
        pl.semaphore_wait(barrier, 2)

        l_ref[...] = jnp.zeros((s_per, 128), jnp.float32)
        out_ref[...] = jnp.zeros((s_per, d), jnp.float32)

        ones_b = jnp.ones((K_TILE, 128), jnp.bfloat16)

        def attn_update(k_src, v_src, n_kt):
            def q_loop(c, _):
                qs = pl.ds(c * Q_TILE, Q_TILE)
                qc = q_ref[qs, :]

                l_acc = l_ref[qs, :]
                acc = out_ref[qs, :]
                for kc in range(n_kt):
                    ks = pl.ds(kc * K_TILE, K_TILE)
                    s = lax.dot_general(
                        qc, k_src[ks, :], (((1,), (1,)), ((), ())),
                        preferred_element_type=jnp.float32,
                    )
                    p = jnp.exp(s).astype(jnp.bfloat16)
                    l_acc = l_acc + lax.dot(
                        p, ones_b, preferred_element_type=jnp.float32)
                    acc = acc + lax.dot(
                        p, v_src[ks, :],
                        preferred_element_type=jnp.float32,
                    )
                l_ref[qs, :] = l_acc
                out_ref[qs, :] = acc
                return 0

            lax.fori_loop(0, n_q, q_loop, 0)

        for h in range(N_DEV):
            if h == 0:
                cw_k, cw_v = k_ref.at[pl.ds(0, half)], v_ref.at[pl.ds(0, half)]
                ccw_k, ccw_v = k_ref.at[pl.ds(half, half)], v_ref.at[pl.ds(half, half)]
            else:
                prev = (h - 1) % 2
                cw_k, cw_v = kcw.at[prev], vcw.at[prev]
                ccw_k, ccw_v = kccw.at[prev], vccw.at[prev]

            rdmas = []
            if h < N_DEV - 1:
                if h >= 1:
                    pl.semaphore_wait(credit_cw, 1)
                    pl.semaphore_wait(credit_ccw, 1)
                slot = h % 2
                for src, dst, sems_s, sems_r, dev in (
                    (cw_k, kcw.at[slot], kcw_s, kcw_r, right),
                    (cw_v, vcw.at[slot], vcw_s, vcw_r, right),
                    (ccw_k, kccw.at[slot], kccw_s, kccw_r, left),
                    (ccw_v, vccw.at[slot], vccw_s, vccw_r, left),
                ):
                    rdma = pltpu.make_async_remote_copy(
                        src_ref=src, dst_ref=dst,
                        send_sem=sems_s.at[h], recv_sem=sems_r.at[h],
                        device_id=(dev,),
                        device_id_type=pl.DeviceIdType.MESH,
                    )
                    rdma.start()
                    rdmas.append(rdma)

            if h == 0:
                attn_update(k_ref, v_ref, s_per // K_TILE)
            else:
                attn_update(cw_k, cw_v, half // K_TILE)
                attn_update(ccw_k, ccw_v, half // K_TILE)

            for rdma in rdmas:
                rdma.wait()
            if h < N_DEV - 2:
                pl.semaphore_signal(credit_cw, inc=1, device_id=(left,),
                                    device_id_type=pl.DeviceIdType.MESH)
                pl.semaphore_signal(credit_ccw, inc=1, device_id=(right,),
                                    device_id_type=pl.DeviceIdType.MESH)

        out_ref[...] = out_ref[...] / l_ref[:, 0:1]

    return pl.pallas_call(
        body,
        out_shape=jax.ShapeDtypeStruct((s_per, d), jnp.float32),
        in_specs=[pl.BlockSpec(memory_space=pltpu.VMEM)] * 3,
        out_specs=pl.BlockSpec(memory_space=pltpu.VMEM),
        scratch_shapes=[
            pltpu.VMEM((2, half, d), jnp.bfloat16),
            pltpu.VMEM((2, half, d), jnp.bfloat16),
            pltpu.VMEM((2, half, d), jnp.bfloat16),
            pltpu.VMEM((2, half, d), jnp.bfloat16),
            pltpu.VMEM((s_per, 128), jnp.float32),
            pltpu.SemaphoreType.DMA((N_DEV - 1,)),
            pltpu.SemaphoreType.DMA((N_DEV - 1,)),
            pltpu.SemaphoreType.DMA((N_DEV - 1,)),
            pltpu.SemaphoreType.DMA((N_DEV - 1,)),
            pltpu.SemaphoreType.DMA((N_DEV - 1,)),
            pltpu.SemaphoreType.DMA((N_DEV - 1,)),
            pltpu.SemaphoreType.DMA((N_DEV - 1,)),
            pltpu.SemaphoreType.DMA((N_DEV - 1,)),
            pltpu.SemaphoreType.REGULAR,
            pltpu.SemaphoreType.REGULAR,
        ],
        compiler_params=pltpu.CompilerParams(
            collective_id=0,
            vmem_limit_bytes=64 * 1024 * 1024,
        ),
    )(qb, kb, vb)
